# Initial kernel scaffold; baseline (speedup 1.0000x reference)
import jax
import jax.numpy as jnp
from jax.experimental import pallas as pl


def kernel(input_ids, table):
    raise NotImplementedError("write your pallas kernel here")


# speedup vs baseline: 1.3068x; 1.3068x over previous
import functools

import jax
import jax.numpy as jnp
from jax import lax
from jax.experimental import pallas as pl
from jax.experimental.pallas import tpu as pltpu
from jax.experimental.pallas import tpu_sc as plsc


def _make_gather(n_workers: int, n_chunks: int, chunk: int, vocab: int, dim: int):
    mesh = plsc.VectorSubcoreMesh(core_axis_name="c", subcore_axis_name="s")

    @functools.partial(
        pl.kernel,
        mesh=mesh,
        out_type=jax.ShapeDtypeStruct((n_workers * n_chunks * chunk, dim), jnp.float32),
        scratch_types=[
            pltpu.VMEM((n_chunks, chunk), jnp.int32),
            pltpu.VMEM((chunk, dim), jnp.float32),
            pltpu.SemaphoreType.DMA,
        ],
        compiler_params=pltpu.CompilerParams(use_tc_tiling_on_sc=False),
    )
    def gather(table_hbm, idx_hbm, out_hbm, idx_v, rows_v, gsem):
        wid = lax.axis_index("s") * 2 + lax.axis_index("c")
        base = wid * n_chunks * chunk
        pltpu.sync_copy(idx_hbm.at[wid], idx_v)

        def body(j, carry):
            pltpu.async_copy(table_hbm.at[idx_v.at[j]], rows_v, gsem).wait()
            pltpu.sync_copy(rows_v, out_hbm.at[pl.ds(base + j * chunk, chunk)])
            return carry

        lax.fori_loop(0, n_chunks, body, 0)

    return gather


def kernel(input_ids, table):
    batch, hist = input_ids.shape
    vocab, dim = table.shape
    n = batch * hist
    n_workers = 32
    chunk = 128
    n_chunks = n // (n_workers * chunk)
    assert n == n_workers * n_chunks * chunk

    idx = input_ids.reshape(n_workers, n_chunks, chunk).astype(jnp.int32)
    gather = _make_gather(n_workers, n_chunks, chunk, vocab, dim)
    out = gather(table, idx)
    return out.reshape(batch, hist, dim)


# speedup vs baseline: 1.4997x; 1.1476x over previous
import functools

import jax
import jax.numpy as jnp
from jax import lax
from jax.experimental import pallas as pl
from jax.experimental.pallas import tpu as pltpu
from jax.experimental.pallas import tpu_sc as plsc


def _make_gather(n_workers: int, n_rounds: int, rpb: int, chunk: int, dim: int):
    rows_per_round = rpb * chunk
    n_rows_w = n_rounds * rows_per_round
    mesh = plsc.VectorSubcoreMesh(core_axis_name="c", subcore_axis_name="s")

    @functools.partial(
        pl.kernel,
        mesh=mesh,
        out_type=jax.ShapeDtypeStruct((n_workers * n_rows_w, dim), jnp.float32),
        scratch_types=[
            pltpu.VMEM((n_rounds * rpb, chunk), jnp.int32),
            pltpu.VMEM((rows_per_round, dim), jnp.float32),
            pltpu.VMEM((rows_per_round, dim), jnp.float32),
            pltpu.SemaphoreType.DMA,
            pltpu.SemaphoreType.DMA,
            pltpu.SemaphoreType.DMA,
        ],
        compiler_params=pltpu.CompilerParams(use_tc_tiling_on_sc=False),
    )
    def gather(table_hbm, idx_hbm, out_hbm, idx_v, buf_a, buf_b, gsem, ssem_a, ssem_b):
        wid = lax.axis_index("s") * 2 + lax.axis_index("c")
        base = wid * n_rows_w
        pltpu.sync_copy(idx_hbm.at[wid], idx_v)

        def issue_gathers(r, buf):
            for b in range(rpb):
                pltpu.async_copy(
                    table_hbm.at[idx_v.at[r * rpb + b]],
                    buf.at[pl.ds(b * chunk, chunk)],
                    gsem,
                )

        def drain_gathers(buf):
            pltpu.make_async_copy(
                table_hbm.at[pl.ds(0, rows_per_round)], buf, gsem
            ).wait()

        def start_store(r, buf, sem):
            pltpu.async_copy(
                buf, out_hbm.at[pl.ds(base + r * rows_per_round, rows_per_round)], sem
            )

        def wait_store(buf, sem):
            pltpu.make_async_copy(
                buf, out_hbm.at[pl.ds(base, rows_per_round)], sem
            ).wait()

        issue_gathers(0, buf_a)

        def body(gg, carry):
            r0 = gg * 2

            @pl.when(gg > 0)
            def _():
                wait_store(buf_b, ssem_b)

            issue_gathers(r0 + 1, buf_b)
            drain_gathers(buf_a)
            start_store(r0, buf_a, ssem_a)

            @pl.when(gg < n_rounds // 2 - 1)
            def _():
                wait_store(buf_a, ssem_a)
                issue_gathers(r0 + 2, buf_a)

            drain_gathers(buf_b)
            start_store(r0 + 1, buf_b, ssem_b)
            return carry

        lax.fori_loop(0, n_rounds // 2, body, 0)
        wait_store(buf_a, ssem_a)
        wait_store(buf_b, ssem_b)

    return gather


def kernel(input_ids, table):
    batch, hist = input_ids.shape
    vocab, dim = table.shape
    n = batch * hist
    n_workers = 32
    chunk = 128
    rpb = 10
    n_rounds = n // (n_workers * rpb * chunk)
    assert n == n_workers * n_rounds * rpb * chunk and n_rounds % 2 == 0

    idx = input_ids.reshape(n_workers, n_rounds * rpb, chunk).astype(jnp.int32)
    gather = _make_gather(n_workers, n_rounds, rpb, chunk, dim)
    out = gather(table, idx)
    return out.reshape(batch, hist, dim)


# speedup vs baseline: 1.5017x; 1.0013x over previous
import functools

import jax
import jax.numpy as jnp
from jax import lax
from jax.experimental import pallas as pl
from jax.experimental.pallas import tpu as pltpu
from jax.experimental.pallas import tpu_sc as plsc


def _make_gather(batch: int, hist: int, dim: int, rows_w: int, nb: int):
    n_workers = batch // rows_w
    n_rounds = rows_w // nb
    splits = []
    off = 0
    while off < hist:
        c = min(128, hist - off)
        splits.append((off, c))
        off += c
    assert all(c % 8 == 0 for _, c in splits)
    mesh = plsc.VectorSubcoreMesh(core_axis_name="c", subcore_axis_name="s")

    @functools.partial(
        pl.kernel,
        mesh=mesh,
        out_type=jax.ShapeDtypeStruct((batch, hist, dim), jnp.float32),
        scratch_types=[
            pltpu.VMEM((rows_w, hist), jnp.int32),
            pltpu.VMEM((nb, hist, dim), jnp.float32),
            pltpu.VMEM((nb, hist, dim), jnp.float32),
            pltpu.SemaphoreType.DMA,
            pltpu.SemaphoreType.DMA,
            pltpu.SemaphoreType.DMA,
        ],
        compiler_params=pltpu.CompilerParams(use_tc_tiling_on_sc=False),
    )
    def gather(table_hbm, idx_hbm, out_hbm, idx_v, buf_a, buf_b, gsem, ssem_a, ssem_b):
        wid = lax.axis_index("s") * 2 + lax.axis_index("c")
        base = wid * rows_w
        pltpu.sync_copy(idx_hbm.at[pl.ds(base, rows_w)], idx_v)

        def issue_gathers(r, buf):
            for b in range(nb):
                row = r * nb + b
                for off, c in splits:
                    pltpu.async_copy(
                        table_hbm.at[idx_v.at[row, pl.ds(off, c)]],
                        buf.at[b, pl.ds(off, c)],
                        gsem,
                    )

        def drain_gathers(buf):
            pltpu.make_async_copy(out_hbm.at[pl.ds(base, nb)], buf, gsem).wait()

        def start_store(r, buf, sem):
            pltpu.async_copy(buf, out_hbm.at[pl.ds(base + r * nb, nb)], sem)

        def wait_store(buf, sem):
            pltpu.make_async_copy(buf, out_hbm.at[pl.ds(base, nb)], sem).wait()

        issue_gathers(0, buf_a)

        def body(gg, carry):
            r0 = gg * 2

            @pl.when(gg > 0)
            def _():
                wait_store(buf_b, ssem_b)

            issue_gathers(r0 + 1, buf_b)
            drain_gathers(buf_a)
            start_store(r0, buf_a, ssem_a)

            @pl.when(gg < n_rounds // 2 - 1)
            def _():
                wait_store(buf_a, ssem_a)
                issue_gathers(r0 + 2, buf_a)

            drain_gathers(buf_b)
            start_store(r0 + 1, buf_b, ssem_b)
            return carry

        lax.fori_loop(0, n_rounds // 2, body, 0)
        wait_store(buf_a, ssem_a)
        wait_store(buf_b, ssem_b)

    return gather


def kernel(input_ids, table):
    batch, hist = input_ids.shape
    vocab, dim = table.shape
    n_workers = 32
    rows_w = batch // n_workers
    nb = 8
    assert batch == n_workers * rows_w and rows_w % (2 * nb) == 0

    gather = _make_gather(batch, hist, dim, rows_w, nb)
    return gather(table, input_ids.astype(jnp.int32))
